# Initial kernel scaffold; baseline (speedup 1.0000x reference)
import jax
import jax.numpy as jnp
from jax.experimental import pallas as pl


def kernel(x, edge_index, edge_index_u, edge_index_v, W_u0, b_u0, W_v0, b_v0, W_uv0, b_uv0, W_u1, b_u1, W_v1, b_v1, W_uv1, b_uv1):
    raise NotImplementedError("write your pallas kernel here")


# speedup vs baseline: 1.3454x; 1.3454x over previous
import functools

import jax
import jax.numpy as jnp
from jax.experimental import pallas as pl

N = 50000
NU = 25000
E = 800000


def _rowmm_body(z_ref, w_ref, b_ref, o_ref):
    o_ref[...] = (
        jnp.dot(z_ref[...], w_ref[...], preferred_element_type=jnp.float32)
        + b_ref[...]
    )


def _rowmm(z, w, b):
    n = z.shape[0]
    blk = 2000
    grid = (n // blk,)
    return pl.pallas_call(
        _rowmm_body,
        grid=grid,
        in_specs=[
            pl.BlockSpec((blk, z.shape[1]), lambda i: (i, 0)),
            pl.BlockSpec((z.shape[1], w.shape[1]), lambda i: (0, 0)),
            pl.BlockSpec((1, w.shape[1]), lambda i: (0, 0)),
        ],
        out_specs=pl.BlockSpec((blk, w.shape[1]), lambda i: (i, 0)),
        out_shape=jax.ShapeDtypeStruct((n, w.shape[1]), jnp.float32),
    )(z, w, b.reshape(1, -1))


def kernel(x, edge_index, edge_index_u, edge_index_v,
           W_u0, b_u0, W_v0, b_v0, W_uv0, b_uv0,
           W_u1, b_u1, W_v1, b_v1, W_uv1, b_uv1):
    x0 = x[:, 0]
    src_e = edge_index[0].astype(jnp.int32)
    dst_e = edge_index[1].astype(jnp.int32)
    src_u = edge_index_u[0].astype(jnp.int32)
    dst_u = edge_index_u[1].astype(jnp.int32)
    src_v = edge_index_v[0].astype(jnp.int32)
    dst_v = edge_index_v[1].astype(jnp.int32)

    def dis_of(dst):
        deg = jnp.ones((N,), jnp.float32).at[dst].add(1.0)
        return deg ** -0.5

    dis_u = dis_of(dst_u)
    dis_v = dis_of(dst_v)
    dis_e = dis_of(dst_e)

    qu = x0 * dis_u
    acc_su = jnp.zeros((N,), jnp.float32).at[dst_u].add(qu[src_u])
    t_u = dis_u * (acc_su + dis_u * x0)
    qv = x0 * dis_v
    acc_sv = jnp.zeros((N,), jnp.float32).at[dst_v].add(qv[src_v])
    t_v = dis_v * (acc_sv + dis_v * x0)
    is_u = jnp.arange(N) < NU
    s = jnp.where(is_u, t_u, t_v)

    alpha = dis_e * s
    beta = dis_e
    sel_v = (src_e >= NU)
    vals = jnp.stack(
        [jnp.where(sel_v, 0.0, alpha[src_e]),
         jnp.where(sel_v, alpha[src_e], 0.0),
         jnp.where(sel_v, 0.0, beta[src_e]),
         jnp.where(sel_v, beta[src_e], 0.0)], axis=1)
    acc4 = jnp.zeros((N, 4), jnp.float32).at[dst_e].add(vals)

    d2 = dis_e * dis_e
    fu = is_u.astype(jnp.float32)
    fv = 1.0 - fu
    c1 = dis_e * acc4[:, 0] + d2 * s * fu
    c2 = dis_e * acc4[:, 1] + d2 * s * fv
    c3 = dis_e * acc4[:, 2] + d2 * fu
    c4 = dis_e * acc4[:, 3] + d2 * fv

    m = jnp.concatenate([W_u0, W_v0, b_u0.reshape(1, -1), b_v0.reshape(1, -1)],
                        axis=0) @ W_uv0
    c4mat = jnp.stack([c1, c2, c3, c4], axis=1)
    x1 = jax.nn.relu(c4mat @ m + b_uv0)

    agg_u = jnp.zeros((N, 64), jnp.float32).at[dst_u].add(
        x1[src_u] * dis_u[src_u, None])
    z_u = dis_u[:, None] * (agg_u + dis_u[:, None] * x1)
    x2_u = _rowmm(z_u, W_u1, b_u1)[:NU]

    agg_v = jnp.zeros((N, 64), jnp.float32).at[dst_v].add(
        x1[src_v] * dis_v[src_v, None])
    z_v = dis_v[:, None] * (agg_v + dis_v[:, None] * x1)
    x2_v = _rowmm(z_v, W_v1, b_v1)[NU:]

    x2 = jnp.concatenate([x2_u, x2_v], axis=0)

    agg_e = jnp.zeros((N, 64), jnp.float32).at[dst_e].add(
        x2[src_e] * dis_e[src_e, None])
    z_e = dis_e[:, None] * (agg_e + dis_e[:, None] * x2)
    return _rowmm(z_e, W_uv1, b_uv1)


# speedup vs baseline: 2.8479x; 2.1168x over previous
import functools

import jax
import jax.numpy as jnp
from jax import lax
from jax.experimental import pallas as pl
from jax.experimental.pallas import tpu as pltpu
from jax.experimental.pallas import tpu_sc as plsc

N = 50000
NU = 25000
E = 800000

NC, NS, L = 2, 16, 16
NPAD = 53248
EPT = E // NS
CHUNK = 10000
NCHUNK = EPT // CHUNK


def _rsqrt16(v):
    i = plsc.bitcast(v, jnp.int32)
    y = plsc.bitcast(jnp.int32(0x5F3759DF) - (i >> 1), jnp.float32)
    for _ in range(3):
        y = y * (1.5 - 0.5 * v * y * y)
    return y


HALF = NPAD // 2
SL = HALF // NS


def _zero1(ref, n):
    def zz(i, _):
        ref[pl.ds(i * L, L)] = jnp.zeros((L,), jnp.float32)
        return 0
    lax.fori_loop(0, n // L, zz, 0)


def _slot_reduce(acc, slots, stg, red, s, post):
    for h in range(2):
        pltpu.sync_copy(acc.at[pl.ds(h * HALF, HALF)],
                        slots.at[pl.ds(s * HALF, HALF)])
        plsc.subcore_barrier()
        _zero1(red, SL)
        for j in range(NS):
            pltpu.sync_copy(slots.at[pl.ds(j * HALF + s * SL, SL)], stg)

            def radd(i, _):
                red[pl.ds(i * L, L)] = (red[pl.ds(i * L, L)]
                                        + stg[pl.ds(i * L, L)])
                return 0
            lax.fori_loop(0, SL // L, radd, 0)
        post(h, red)
        plsc.subcore_barrier()


def _deg_body(du, dv, de, o_u, o_v, o_e, acc, buf, stg, red, slots):
    c = lax.axis_index("c")
    s = lax.axis_index("s")
    ones = jnp.full((L,), 1.0, jnp.float32)

    for dst_hbm, out_hbm in ((du, o_u), (dv, o_v), (de, o_e)):
        _zero1(acc, NPAD)
        for k in range(NCHUNK):
            pltpu.sync_copy(dst_hbm.at[pl.ds(s * EPT + k * CHUNK, CHUNK)], buf)

            def step(i, _):
                idx = buf[pl.ds(i * L, L)]
                plsc.addupdate_scatter(acc, [idx], ones)
                return 0
            lax.fori_loop(0, CHUNK // L, step, 0)

        def post(h, red):
            def dz(i, _):
                red[pl.ds(i * L, L)] = _rsqrt16(red[pl.ds(i * L, L)] + 1.0)
                return 0
            lax.fori_loop(0, SL // L, dz, 0)

            @pl.when(h == c)
            def _():
                pltpu.sync_copy(red, out_hbm.at[pl.ds(h * HALF + s * SL, SL)])

        _slot_reduce(acc, slots, stg, red, s, post)


def _sc_dis(du, dv, de):
    f = pl.kernel(
        _deg_body,
        out_type=[jax.ShapeDtypeStruct((NPAD,), jnp.float32)] * 3,
        mesh=plsc.VectorSubcoreMesh(core_axis_name="c", subcore_axis_name="s",
                                    num_cores=NC, num_subcores=NS),
        compiler_params=pltpu.CompilerParams(needs_layout_passes=False),
        scratch_types=[
            pltpu.VMEM((NPAD,), jnp.float32),
            pltpu.VMEM((CHUNK,), jnp.int32),
            pltpu.VMEM((SL,), jnp.float32),
            pltpu.VMEM((SL,), jnp.float32),
            pltpu.VMEM_SHARED((NS * HALF,), jnp.float32),
        ],
    )
    o_u, o_v, o_e = f(du, dv, de)
    return o_u[:N], o_v[:N], o_e[:N]


def _rowmm_body(z_ref, w_ref, b_ref, o_ref):
    o_ref[...] = (
        jnp.dot(z_ref[...], w_ref[...], preferred_element_type=jnp.float32)
        + b_ref[...]
    )


def _rowmm(z, w, b):
    n = z.shape[0]
    blk = 2000
    grid = (n // blk,)
    return pl.pallas_call(
        _rowmm_body,
        grid=grid,
        in_specs=[
            pl.BlockSpec((blk, z.shape[1]), lambda i: (i, 0)),
            pl.BlockSpec((z.shape[1], w.shape[1]), lambda i: (0, 0)),
            pl.BlockSpec((1, w.shape[1]), lambda i: (0, 0)),
        ],
        out_specs=pl.BlockSpec((blk, w.shape[1]), lambda i: (i, 0)),
        out_shape=jax.ShapeDtypeStruct((n, w.shape[1]), jnp.float32),
    )(z, w, b.reshape(1, -1))


D = 64
RACC = 13440
TRASH = RACC - 1
CB = 2048
B = 128
EPAD = 819200
EPP = EPAD // NS
ZR = 56


def _row_pass_body(phases, d):
    def body(src_h, dst_h, table, z_h, out, sbuf, dbuf, gidx, sidx, rbuf,
             acc, sem):
        c = lax.axis_index("c")
        s = lax.axis_index("s")
        iota = lax.iota(jnp.int32, L)
        zeros16 = jnp.zeros((L,), jnp.int32)

        for base0, base1, R in phases:
            base = jnp.where(c == 0, base0, base1).astype(jnp.int32)
            pltpu.sync_copy(z_h, acc.at[pl.ds(s * (RACC // NS), RACC // NS)])
            plsc.subcore_barrier()

            for k in range(EPP // CB):
                eoff = s * EPP + k * CB
                pltpu.sync_copy(src_h.at[pl.ds(eoff, CB)], sbuf)
                pltpu.sync_copy(dst_h.at[pl.ds(eoff, CB)], dbuf)

                def step(i, cnt):
                    sv = sbuf[pl.ds(i * L, L)]
                    dl = dbuf[pl.ds(i * L, L)] - base
                    m = (dl >= 0) & (dl < R)
                    mi = jnp.where(m, 1, 0).astype(jnp.int32)
                    offs = cnt + plsc.cumsum(mi) - mi
                    ob, oc = offs >> 7, offs & (B - 1)
                    plsc.store_scatter(gidx, [ob, oc], sv, mask=m)
                    plsc.store_scatter(sidx, [ob, oc], dl, mask=m)
                    return cnt + plsc.all_reduce_population_count(m)
                cnt = lax.fori_loop(0, CB // L, step, zeros16)

                tgt = (cnt + (B - 1)) & jnp.int32(-B)
                for t in range(8):
                    offs = cnt + iota + t * L
                    m = offs < tgt
                    ob, oc = offs >> 7, offs & (B - 1)
                    plsc.store_scatter(gidx, [ob, oc], zeros16, mask=m)
                    plsc.store_scatter(sidx, [ob, oc],
                                       jnp.full((L,), TRASH, jnp.int32),
                                       mask=m)

                nb = jnp.max(tgt) >> 7

                def flush(b, _):
                    pltpu.async_copy(table.at[gidx.at[b]], rbuf, sem).wait()
                    pltpu.sync_copy(rbuf, acc.at[sidx.at[b]], add=True)
                    return 0
                lax.fori_loop(0, nb, flush, 0)
            plsc.subcore_barrier()

            rpt = R // NS
            pltpu.sync_copy(acc.at[pl.ds(s * rpt, rpt)],
                            out.at[pl.ds(base + s * rpt, rpt)])
            plsc.subcore_barrier()
    return body


def _sc_row_pass(src, dst, table, phases, d=D):
    f = pl.kernel(
        _row_pass_body(tuple(phases), d),
        out_type=jax.ShapeDtypeStruct((NPAD, d), jnp.float32),
        mesh=plsc.VectorSubcoreMesh(core_axis_name="c", subcore_axis_name="s",
                                    num_cores=NC, num_subcores=NS),
        compiler_params=pltpu.CompilerParams(needs_layout_passes=False,
                                             use_tc_tiling_on_sc=False),
        scratch_types=[
            pltpu.VMEM((CB,), jnp.int32),
            pltpu.VMEM((CB,), jnp.int32),
            pltpu.VMEM((CB // B, B), jnp.int32),
            pltpu.VMEM((CB // B, B), jnp.int32),
            pltpu.VMEM((B, d), jnp.float32),
            pltpu.VMEM_SHARED((RACC, d), jnp.float32),
            pltpu.SemaphoreType.DMA,
        ],
    )
    return f(src, dst, table, jnp.zeros((RACC // NS, d), jnp.float32))


def kernel(x, edge_index, edge_index_u, edge_index_v,
           W_u0, b_u0, W_v0, b_v0, W_uv0, b_uv0,
           W_u1, b_u1, W_v1, b_v1, W_uv1, b_uv1):
    x0 = x[:, 0]
    src_e = edge_index[0].astype(jnp.int32)
    dst_e = edge_index[1].astype(jnp.int32)
    src_u = edge_index_u[0].astype(jnp.int32)
    dst_u = edge_index_u[1].astype(jnp.int32)
    src_v = edge_index_v[0].astype(jnp.int32)
    dst_v = edge_index_v[1].astype(jnp.int32)

    sfill = jnp.zeros((EPAD - E,), jnp.int32)
    dfill = jnp.full((EPAD - E,), -1, jnp.int32)

    def padded(a, fill):
        return jnp.concatenate([a, fill])

    def padtab(t):
        return jnp.pad(t, ((0, NPAD - N), (0, 0)))


    dis_u, dis_v, dis_e = _sc_dis(dst_u, dst_v, dst_e)

    psrc_u, pdst_u = padded(src_u, sfill), padded(dst_u, dfill)
    psrc_v, pdst_v = padded(src_v, sfill), padded(dst_v, dfill)
    psrc_e, pdst_e = padded(src_e, sfill), padded(dst_e, dfill)

    acc_su = jnp.zeros((N,), jnp.float32).at[dst_u].add((x0 * dis_u)[src_u])
    t_u = dis_u * (acc_su + dis_u * x0)
    acc_sv = jnp.zeros((N,), jnp.float32).at[dst_v].add((x0 * dis_v)[src_v])
    t_v = dis_v * (acc_sv + dis_v * x0)
    is_u = jnp.arange(N) < NU
    s = jnp.where(is_u, t_u, t_v)

    alpha = dis_e * s
    beta = dis_e
    zn = jnp.zeros((N,), jnp.float32)
    t4 = jnp.stack([jnp.where(is_u, alpha, zn), jnp.where(is_u, zn, alpha),
                    jnp.where(is_u, beta, zn), jnp.where(is_u, zn, beta)],
                   axis=1)
    acc4 = jnp.zeros((N, 4), jnp.float32).at[dst_e].add(t4[src_e])

    d2 = dis_e * dis_e
    fu = is_u.astype(jnp.float32)
    fv = 1.0 - fu
    c1 = dis_e * acc4[:, 0] + d2 * s * fu
    c2 = dis_e * acc4[:, 1] + d2 * s * fv
    c3 = dis_e * acc4[:, 2] + d2 * fu
    c4 = dis_e * acc4[:, 3] + d2 * fv

    m = jnp.concatenate([W_u0, W_v0, b_u0.reshape(1, -1), b_v0.reshape(1, -1)],
                        axis=0) @ W_uv0
    c4mat = jnp.stack([c1, c2, c3, c4], axis=1)
    x1 = jax.nn.relu(_rowmm(c4mat, m, b_uv0))

    agg_u = _sc_row_pass(psrc_u, pdst_u, padtab(x1 * dis_u[:, None]),
                         [(0, 13312, 13312)])[:N]
    z_u = dis_u[:, None] * (agg_u + dis_u[:, None] * x1)
    x2_u = _rowmm(z_u, W_u1, b_u1)[:NU]

    agg_v = _sc_row_pass(psrc_v, pdst_v, padtab(x1 * dis_v[:, None]),
                         [(25000, 37544, 12544)])[:N]
    z_v = dis_v[:, None] * (agg_v + dis_v[:, None] * x1)
    x2_v = _rowmm(z_v, W_v1, b_v1)[NU:]

    x2 = jnp.concatenate([x2_u, x2_v], axis=0)

    agg_e = _sc_row_pass(psrc_e, pdst_e, padtab(x2 * dis_e[:, None]),
                         [(0, 26624, 13312), (13312, 39936, 13312)])[:N]
    z_e = dis_e[:, None] * (agg_e + dis_e[:, None] * x2)
    return _rowmm(z_e, W_uv1, b_uv1)


# speedup vs baseline: 5.7669x; 2.0250x over previous
import functools

import jax
import jax.numpy as jnp
from jax import lax
from jax.experimental import pallas as pl
from jax.experimental.pallas import tpu as pltpu
from jax.experimental.pallas import tpu_sc as plsc

N = 50000
NU = 25000
E = 800000

NC, NS, L = 2, 16, 16
NPAD = 53248
EPT = E // NS
CHUNK = 10000
NCHUNK = EPT // CHUNK


def _rsqrt16(v):
    i = plsc.bitcast(v, jnp.int32)
    y = plsc.bitcast(jnp.int32(0x5F3759DF) - (i >> 1), jnp.float32)
    for _ in range(3):
        y = y * (1.5 - 0.5 * v * y * y)
    return y


HALF = NPAD // 2
SL = HALF // NS


def _zero1(ref, n):
    def zz(i, _):
        ref[pl.ds(i * L, L)] = jnp.zeros((L,), jnp.float32)
        return 0
    lax.fori_loop(0, n // L, zz, 0)


def _slot_reduce(acc, slots, stg, red, s, post):
    for h in range(2):
        pltpu.sync_copy(acc.at[pl.ds(h * HALF, HALF)],
                        slots.at[pl.ds(s * HALF, HALF)])
        plsc.subcore_barrier()
        _zero1(red, SL)
        for j in range(NS):
            pltpu.sync_copy(slots.at[pl.ds(j * HALF + s * SL, SL)], stg)

            def radd(i, _):
                red[pl.ds(i * L, L)] = (red[pl.ds(i * L, L)]
                                        + stg[pl.ds(i * L, L)])
                return 0
            lax.fori_loop(0, SL // L, radd, 0)
        post(h, red)
        plsc.subcore_barrier()


def _deg_body(du, dv, de, o_u, o_v, o_e, acc, buf, stg, red, slots):
    c = lax.axis_index("c")
    s = lax.axis_index("s")
    ones = jnp.full((L,), 1.0, jnp.float32)

    for dst_hbm, out_hbm in ((du, o_u), (dv, o_v), (de, o_e)):
        _zero1(acc, NPAD)
        for k in range(NCHUNK):
            pltpu.sync_copy(dst_hbm.at[pl.ds(s * EPT + k * CHUNK, CHUNK)], buf)

            def step(i, _):
                idx = buf[pl.ds(i * L, L)]
                plsc.addupdate_scatter(acc, [idx], ones)
                return 0
            lax.fori_loop(0, CHUNK // L, step, 0)

        def post(h, red):
            def dz(i, _):
                red[pl.ds(i * L, L)] = _rsqrt16(red[pl.ds(i * L, L)] + 1.0)
                return 0
            lax.fori_loop(0, SL // L, dz, 0)

            @pl.when(h == c)
            def _():
                pltpu.sync_copy(red, out_hbm.at[pl.ds(h * HALF + s * SL, SL)])

        _slot_reduce(acc, slots, stg, red, s, post)


def _sc_dis(du, dv, de):
    f = pl.kernel(
        _deg_body,
        out_type=[jax.ShapeDtypeStruct((NPAD,), jnp.float32)] * 3,
        mesh=plsc.VectorSubcoreMesh(core_axis_name="c", subcore_axis_name="s",
                                    num_cores=NC, num_subcores=NS),
        compiler_params=pltpu.CompilerParams(needs_layout_passes=False),
        scratch_types=[
            pltpu.VMEM((NPAD,), jnp.float32),
            pltpu.VMEM((CHUNK,), jnp.int32),
            pltpu.VMEM((SL,), jnp.float32),
            pltpu.VMEM((SL,), jnp.float32),
            pltpu.VMEM_SHARED((NS * HALF,), jnp.float32),
        ],
    )
    o_u, o_v, o_e = f(du, dv, de)
    return o_u[:N], o_v[:N], o_e[:N]


def _rowmm_body(z_ref, w_ref, b_ref, o_ref):
    o_ref[...] = (
        jnp.dot(z_ref[...], w_ref[...], preferred_element_type=jnp.float32)
        + b_ref[...]
    )


def _rowmm(z, w, b):
    n = z.shape[0]
    blk = 2000
    grid = (n // blk,)
    return pl.pallas_call(
        _rowmm_body,
        grid=grid,
        in_specs=[
            pl.BlockSpec((blk, z.shape[1]), lambda i: (i, 0)),
            pl.BlockSpec((z.shape[1], w.shape[1]), lambda i: (0, 0)),
            pl.BlockSpec((1, w.shape[1]), lambda i: (0, 0)),
        ],
        out_specs=pl.BlockSpec((blk, w.shape[1]), lambda i: (i, 0)),
        out_shape=jax.ShapeDtypeStruct((n, w.shape[1]), jnp.float32),
    )(z, w, b.reshape(1, -1))


D = 64
RACC = 13440
TRASH = RACC - 1
CB = 2048
B = 128
EPAD = 819200
EPP = EPAD // NS
ZR = 56


def _row_pass_body(phases, d):
    def body(src_h, dst_h, table, z_h, out, sbuf, dbuf, gidx, sidx, rbuf,
             acc, sem):
        c = lax.axis_index("c")
        s = lax.axis_index("s")
        iota = lax.iota(jnp.int32, L)
        zeros16 = jnp.zeros((L,), jnp.int32)

        for base0, base1, R in phases:
            base = jnp.where(c == 0, base0, base1).astype(jnp.int32)
            pltpu.sync_copy(z_h, acc.at[pl.ds(s * (RACC // NS), RACC // NS)])
            plsc.subcore_barrier()

            for k in range(EPP // CB):
                eoff = s * EPP + k * CB
                pltpu.sync_copy(src_h.at[pl.ds(eoff, CB)], sbuf)
                pltpu.sync_copy(dst_h.at[pl.ds(eoff, CB)], dbuf)

                def step(i, cnt):
                    sv = sbuf[pl.ds(i * L, L)]
                    dl = dbuf[pl.ds(i * L, L)] - base
                    m = (dl >= 0) & (dl < R)
                    mi = jnp.where(m, 1, 0).astype(jnp.int32)
                    offs = cnt + plsc.cumsum(mi) - mi
                    ob, oc = offs >> 7, offs & (B - 1)
                    plsc.store_scatter(gidx, [ob, oc], sv, mask=m)
                    plsc.store_scatter(sidx, [ob, oc], dl, mask=m)
                    return cnt + plsc.all_reduce_population_count(m)
                cnt = lax.fori_loop(0, CB // L, step, zeros16)

                tgt = (cnt + (B - 1)) & jnp.int32(-B)
                for t in range(8):
                    offs = cnt + iota + t * L
                    m = offs < tgt
                    ob, oc = offs >> 7, offs & (B - 1)
                    plsc.store_scatter(gidx, [ob, oc], zeros16, mask=m)
                    plsc.store_scatter(sidx, [ob, oc],
                                       jnp.full((L,), TRASH, jnp.int32),
                                       mask=m)

                nb = jnp.max(tgt) >> 7

                def flush(b, _):
                    pltpu.async_copy(table.at[gidx.at[b]], rbuf, sem).wait()
                    pltpu.sync_copy(rbuf, acc.at[sidx.at[b]], add=True)
                    return 0
                lax.fori_loop(0, nb, flush, 0)
            plsc.subcore_barrier()

            rpt = R // NS
            pltpu.sync_copy(acc.at[pl.ds(s * rpt, rpt)],
                            out.at[pl.ds(base + s * rpt, rpt)])
            plsc.subcore_barrier()
    return body


def _sc_row_pass(src, dst, table, phases, d=D):
    f = pl.kernel(
        _row_pass_body(tuple(phases), d),
        out_type=jax.ShapeDtypeStruct((NPAD, d), jnp.float32),
        mesh=plsc.VectorSubcoreMesh(core_axis_name="c", subcore_axis_name="s",
                                    num_cores=NC, num_subcores=NS),
        compiler_params=pltpu.CompilerParams(needs_layout_passes=False,
                                             use_tc_tiling_on_sc=False),
        scratch_types=[
            pltpu.VMEM((CB,), jnp.int32),
            pltpu.VMEM((CB,), jnp.int32),
            pltpu.VMEM((CB // B, B), jnp.int32),
            pltpu.VMEM((CB // B, B), jnp.int32),
            pltpu.VMEM((B, d), jnp.float32),
            pltpu.VMEM_SHARED((RACC, d), jnp.float32),
            pltpu.SemaphoreType.DMA,
        ],
    )
    return f(src, dst, table, jnp.zeros((RACC // NS, d), jnp.float32))


def kernel(x, edge_index, edge_index_u, edge_index_v,
           W_u0, b_u0, W_v0, b_v0, W_uv0, b_uv0,
           W_u1, b_u1, W_v1, b_v1, W_uv1, b_uv1):
    x0 = x[:, 0]
    src_e = edge_index[0].astype(jnp.int32)
    dst_e = edge_index[1].astype(jnp.int32)
    src_u = edge_index_u[0].astype(jnp.int32)
    dst_u = edge_index_u[1].astype(jnp.int32)
    src_v = edge_index_v[0].astype(jnp.int32)
    dst_v = edge_index_v[1].astype(jnp.int32)

    sfill = jnp.zeros((EPAD - E,), jnp.int32)
    dfill = jnp.full((EPAD - E,), -1, jnp.int32)

    def padded(a, fill):
        return jnp.concatenate([a, fill])

    def padtab(t):
        return jnp.pad(t, ((0, NPAD - N), (0, D - t.shape[1])))


    dis_u, dis_v, dis_e = _sc_dis(dst_u, dst_v, dst_e)

    psrc_u, pdst_u = padded(src_u, sfill), padded(dst_u, dfill)
    psrc_v, pdst_v = padded(src_v, sfill), padded(dst_v, dfill)
    psrc_e, pdst_e = padded(src_e, sfill), padded(dst_e, dfill)

    acc_su = _sc_row_pass(psrc_u, pdst_u, padtab((x0 * dis_u)[:, None]),
                          [(0, 13312, 13312)])[:N, 0]
    t_u = dis_u * (acc_su + dis_u * x0)
    acc_sv = _sc_row_pass(psrc_v, pdst_v, padtab((x0 * dis_v)[:, None]),
                          [(25000, 37544, 12544)])[:N, 0]
    t_v = dis_v * (acc_sv + dis_v * x0)
    is_u = jnp.arange(N) < NU
    s = jnp.where(is_u, t_u, t_v)

    alpha = dis_e * s
    beta = dis_e
    zn = jnp.zeros((N,), jnp.float32)
    t4 = jnp.stack([jnp.where(is_u, alpha, zn), jnp.where(is_u, zn, alpha),
                    jnp.where(is_u, beta, zn), jnp.where(is_u, zn, beta)],
                   axis=1)
    acc4 = _sc_row_pass(psrc_e, pdst_e, padtab(t4),
                        [(0, 26624, 13312), (13312, 39936, 13312)])[:N, :4]

    d2 = dis_e * dis_e
    fu = is_u.astype(jnp.float32)
    fv = 1.0 - fu
    c1 = dis_e * acc4[:, 0] + d2 * s * fu
    c2 = dis_e * acc4[:, 1] + d2 * s * fv
    c3 = dis_e * acc4[:, 2] + d2 * fu
    c4 = dis_e * acc4[:, 3] + d2 * fv

    m = jnp.concatenate([W_u0, W_v0, b_u0.reshape(1, -1), b_v0.reshape(1, -1)],
                        axis=0) @ W_uv0
    c4mat = jnp.stack([c1, c2, c3, c4], axis=1)
    x1 = jax.nn.relu(_rowmm(c4mat, m, b_uv0))

    agg_u = _sc_row_pass(psrc_u, pdst_u, padtab(x1 * dis_u[:, None]),
                         [(0, 13312, 13312)])[:N]
    z_u = dis_u[:, None] * (agg_u + dis_u[:, None] * x1)
    x2_u = _rowmm(z_u, W_u1, b_u1)[:NU]

    agg_v = _sc_row_pass(psrc_v, pdst_v, padtab(x1 * dis_v[:, None]),
                         [(25000, 37544, 12544)])[:N]
    z_v = dis_v[:, None] * (agg_v + dis_v[:, None] * x1)
    x2_v = _rowmm(z_v, W_v1, b_v1)[NU:]

    x2 = jnp.concatenate([x2_u, x2_v], axis=0)

    agg_e = _sc_row_pass(psrc_e, pdst_e, padtab(x2 * dis_e[:, None]),
                         [(0, 26624, 13312), (13312, 39936, 13312)])[:N]
    z_e = dis_e[:, None] * (agg_e + dis_e[:, None] * x2)
    return _rowmm(z_e, W_uv1, b_uv1)
